# baseline (speedup 1.0000x reference)
import jax
import jax.numpy as jnp
from jax.experimental import pallas as pl
from jax.experimental.pallas import tpu as pltpu

_S = 16
_PACK = 8
_BLOCK_I = 64
_N_LANES = 512
_ROWS = 4096
_GRID = _ROWS // _BLOCK_I


def _inner(indices, a, bcol, m_ref, x_ref, o_ref):
    (i,) = indices
    mf = m_ref[i].astype(jnp.float32)
    x = x_ref[...]
    for k in range(_BLOCK_I // _PACK):
        r = _PACK * k
        xk = x[r : r + _PACK].reshape(_PACK * _S, _N_LANES)
        y = jnp.dot(a, xk, preferred_element_type=jnp.float32) + bcol
        me = jnp.repeat(mf[r : r + _PACK, :], _S, axis=0)
        o_ref[r : r + _PACK] = (y * me).reshape(_PACK, _S, _N_LANES)


def _masked_linear_kernel(x_hbm, m_ref, a_ref, b_ref, o_hbm):
    a = a_ref[...]
    bcol = b_ref[...][:, 0:1]

    pipeline = pltpu.emit_pipeline(
        lambda idx, x_ref, o_ref: _inner(idx, a, bcol, m_ref, x_ref, o_ref),
        grid=(_GRID,),
        in_specs=[
            pl.BlockSpec(
                (_BLOCK_I, _S, _N_LANES),
                lambda i: (i, 0, 0),
                pipeline_mode=pl.Buffered(buffer_count=8),
            ),
        ],
        out_specs=[
            pl.BlockSpec((_BLOCK_I, _S, _N_LANES), lambda i: (i, 0, 0)),
        ],
        _explicit_indices=True,
    )
    pipeline(x_hbm, o_hbm)


@jax.jit
def kernel(stacks, mask, W, bias):
    b, n, _, s = stacks.shape
    h = W.shape[0]
    rows = b * n

    xt = jnp.transpose(stacks, (0, 1, 3, 2)).reshape(rows, s, n)
    m3 = mask.reshape(_GRID, _BLOCK_I, n)

    a_big = jnp.kron(jnp.eye(_PACK, dtype=W.dtype), W)
    b_big = jnp.tile(jnp.tile(bias, _PACK)[:, None], (1, _PACK * h))

    out = pl.pallas_call(
        _masked_linear_kernel,
        in_specs=[
            pl.BlockSpec(memory_space=pl.ANY),
            pl.BlockSpec(memory_space=pltpu.MemorySpace.VMEM),
            pl.BlockSpec(memory_space=pltpu.MemorySpace.VMEM),
            pl.BlockSpec(memory_space=pltpu.MemorySpace.VMEM),
        ],
        out_specs=pl.BlockSpec(memory_space=pl.ANY),
        out_shape=jax.ShapeDtypeStruct((rows, h, n), jnp.float32),
        compiler_params=pltpu.CompilerParams(
            vmem_limit_bytes=100 * 1024 * 1024,
        ),
    )(xt, m3, a_big, b_big)

    return jnp.transpose(out.reshape(b, n, h, n), (0, 1, 3, 2))


# speedup vs baseline: 1.0433x; 1.0433x over previous
import jax
import jax.numpy as jnp
from jax.experimental import pallas as pl
from jax.experimental.pallas import tpu as pltpu

_S = 16
_PACK = 8
_BLOCK_I = 64
_N_LANES = 512
_ROWS = 4096


def _inner(a, bcol, x_ref, m_ref, o_ref):
    mf = m_ref[...].astype(jnp.float32)
    x = x_ref[...]
    for k in range(_BLOCK_I // _PACK):
        r = _PACK * k
        xk = x[r : r + _PACK].reshape(_PACK * _S, _N_LANES)
        y = jnp.dot(a, xk, preferred_element_type=jnp.float32) + bcol
        me = jnp.repeat(mf[r : r + _PACK, :], _S, axis=0)
        o_ref[r : r + _PACK] = (y * me).reshape(_PACK, _S, _N_LANES)


def _masked_linear_kernel(x_hbm, m_hbm, a_ref, b_ref, o_hbm):
    a = a_ref[...]
    bcol = b_ref[...][:, 0:1]

    pipeline = pltpu.emit_pipeline(
        lambda x_ref, m_ref, o_ref: _inner(a, bcol, x_ref, m_ref, o_ref),
        grid=(_ROWS // _BLOCK_I,),
        in_specs=[
            pl.BlockSpec(
                (_BLOCK_I, _S, _N_LANES),
                lambda i: (i, 0, 0),
                pipeline_mode=pl.Buffered(buffer_count=8),
            ),
            pl.BlockSpec(
                (_BLOCK_I, _N_LANES),
                lambda i: (i, 0),
                pipeline_mode=pl.Buffered(buffer_count=8),
            ),
        ],
        out_specs=[
            pl.BlockSpec((_BLOCK_I, _S, _N_LANES), lambda i: (i, 0, 0)),
        ],
    )
    pipeline(x_hbm, m_hbm, o_hbm)


@jax.jit
def kernel(stacks, mask, W, bias):
    b, n, _, s = stacks.shape
    h = W.shape[0]
    rows = b * n

    xt = jnp.transpose(stacks, (0, 1, 3, 2)).reshape(rows, s, n)
    m2 = mask.view(jnp.int8).reshape(rows, n)

    a_big = jnp.kron(jnp.eye(_PACK, dtype=W.dtype), W)
    b_big = jnp.tile(jnp.tile(bias, _PACK)[:, None], (1, _PACK * h))

    out = pl.pallas_call(
        _masked_linear_kernel,
        in_specs=[
            pl.BlockSpec(memory_space=pl.ANY),
            pl.BlockSpec(memory_space=pl.ANY),
            pl.BlockSpec(memory_space=pltpu.MemorySpace.VMEM),
            pl.BlockSpec(memory_space=pltpu.MemorySpace.VMEM),
        ],
        out_specs=pl.BlockSpec(memory_space=pl.ANY),
        out_shape=jax.ShapeDtypeStruct((rows, h, n), jnp.float32),
        compiler_params=pltpu.CompilerParams(
            vmem_limit_bytes=100 * 1024 * 1024,
        ),
    )(xt, m2, a_big, b_big)

    return jnp.transpose(out.reshape(b, n, h, n), (0, 1, 3, 2))
